# baseline (speedup 1.0000x reference)
import functools

import jax
import jax.numpy as jnp
import numpy as np
from jax.experimental import pallas as pl
from jax.experimental.pallas import tpu as pltpu

_N = 10000
_NE = 160000
_OMEGA = 100.0
_BETA = 0.2
_H = 16
_C = 16
_BLK = 1024


def _edge_mlp_body(xin_ref, v_ref, w0_ref, b0_ref, w1_ref, b1_ref,
                   w2_ref, b2_ref, sel_ref, msg_ref):
    xin = xin_ref[...]
    h = jnp.dot(xin, w0_ref[...], preferred_element_type=jnp.float32)
    h = jnp.sin(_OMEGA * (h + b0_ref[...]))
    h = jnp.dot(h, w1_ref[...], preferred_element_type=jnp.float32)
    h = jnp.sin(_OMEGA * (h + b1_ref[...]))
    h = jnp.dot(h, w2_ref[...], preferred_element_type=jnp.float32)
    h = jnp.sin(_OMEGA * (h + b2_ref[...]))
    vt = jnp.tile(v_ref[...], (1, _C))
    msg_ref[...] = jnp.dot(h * vt, sel_ref[...],
                           preferred_element_type=jnp.float32)


def _edge_mlp(xin, v, p, e_pad):
    w0 = p['s0W']
    w0bd = np.zeros((64, 16 * _C), np.float32)
    w1bd = np.zeros((16 * _C, 16 * _C), np.float32)
    w2bd = np.zeros((16 * _C, 16 * _C), np.float32)
    sel = np.zeros((16 * _C, _C), np.float32)
    w0bd = jnp.zeros((64, 16 * _C), jnp.float32)
    w1bd = jnp.zeros((16 * _C, 16 * _C), jnp.float32)
    w2bd = jnp.zeros((16 * _C, 16 * _C), jnp.float32)
    sel = jnp.zeros((16 * _C, _C), jnp.float32)
    for c in range(_C):
        w0bd = w0bd.at[c * 4:(c + 1) * 4, c * 16:(c + 1) * 16].set(w0.T)
        w1bd = w1bd.at[c * 16:(c + 1) * 16, c * 16:(c + 1) * 16].set(p['s1W'].T)
        w2bd = w2bd.at[c * 16:(c + 1) * 16, c * 16:(c + 1) * 16].set(p['s2W'].T)
        sel = sel.at[c * 16:(c + 1) * 16, c].set(1.0)
    b0t = jnp.tile(p['s0b'], _C)
    b1t = jnp.tile(p['s1b'], _C)
    b2t = jnp.tile(p['s2b'], _C)

    grid = (e_pad // _BLK,)
    full = lambda i: (0, 0)
    msg = pl.pallas_call(
        _edge_mlp_body,
        grid=grid,
        in_specs=[
            pl.BlockSpec((_BLK, 64), lambda i: (i, 0)),
            pl.BlockSpec((_BLK, _H), lambda i: (i, 0)),
            pl.BlockSpec((64, 16 * _C), full),
            pl.BlockSpec((1, 16 * _C), lambda i: (0, 0)),
            pl.BlockSpec((16 * _C, 16 * _C), full),
            pl.BlockSpec((1, 16 * _C), lambda i: (0, 0)),
            pl.BlockSpec((16 * _C, 16 * _C), full),
            pl.BlockSpec((1, 16 * _C), lambda i: (0, 0)),
            pl.BlockSpec((16 * _C, _C), full),
        ],
        out_specs=pl.BlockSpec((_BLK, _C), lambda i: (i, 0)),
        out_shape=jax.ShapeDtypeStruct((e_pad, _C), jnp.float32),
    )(xin, v, w0bd, b0t[None], w1bd, b1t[None], w2bd, b2t[None], sel)
    return msg


def _conv_msgs(xin, src, p, x_parts, e_pad):
    linW, linb = p['linW'], p['linb']
    xw = sum(jnp.dot(xp, linW[o:o + xp.shape[1]]) for xp, o in x_parts)
    xb = sum(jnp.dot(xp, linb[o:o + xp.shape[1]]) for xp, o in x_parts)
    v = xw[src]
    b = xb[src]
    msg = _edge_mlp(xin, v, p, e_pad)
    return msg + b[:, None]


def kernel(x, pos, params, edge_index):
    src0, dst0 = edge_index[0], edge_index[1]
    loops = jnp.arange(_N, dtype=edge_index.dtype)
    src = jnp.concatenate((src0, loops))
    dst = jnp.concatenate((dst0, loops))
    maskf = jnp.concatenate(((src0 != dst0).astype(jnp.float32),
                             jnp.ones((_N,), jnp.float32)))
    e = src.shape[0]
    e_pad = ((e + _BLK - 1) // _BLK) * _BLK
    pad = e_pad - e
    src_p = jnp.pad(src, (0, pad))
    rel = pos[dst] - pos[src]
    sq = jnp.sum(rel * rel, axis=1)
    rho = jnp.where(sq > 0, jnp.sqrt(jnp.where(sq > 0, sq, 1.0)), 0.0)
    x0, y0, z0c = rel[:, 0], rel[:, 1], rel[:, 2]
    xy_zero = (x0 == 0) & (y0 == 0)
    theta = jnp.arctan2(y0, jnp.where(xy_zero, 1.0, x0))
    phi = jnp.arcsin(z0c / jnp.where(rho == 0, 1.0, rho))
    ch = jnp.arange(_C, dtype=jnp.float32)
    feats4 = jnp.stack((rho, theta / jnp.pi, phi / jnp.pi), axis=1)
    xin = jnp.concatenate(
        (jnp.broadcast_to(feats4[:, None, :], (e, _C, 3)),
         jnp.broadcast_to(ch[None, :, None], (e, _C, 1))), axis=2
    ).reshape(e, 64)
    xin = jnp.pad(xin, ((0, pad), (0, 0)))

    p1, p2, p3 = params['conv1'], params['conv2'], params['conv3']

    def agg_of(msg, p):
        m = (msg[:e] * maskf[:, None])
        return jnp.zeros((_N, _C), jnp.float32).at[dst].add(m) + p['bias']

    x1 = jax.nn.selu(agg_of(_conv_msgs(xin, src_p, p1, [(x, 0)], e_pad), p1))
    x2 = jax.nn.selu(agg_of(
        _conv_msgs(xin, src_p, p2, [(x, 0), (x1, 16)], e_pad), p2))
    x3 = jax.nn.selu(agg_of(
        _conv_msgs(xin, src_p, p3, [(x, 0), (x1, 16), (x2, 32)], e_pad), p3))
    return x + _BETA * x3


# speedup vs baseline: 1.3393x; 1.3393x over previous
import functools

import jax
import jax.numpy as jnp
import numpy as np
from jax.experimental import pallas as pl
from jax.experimental.pallas import tpu as pltpu

_N = 10000
_NE = 160000
_OMEGA = 100.0
_BETA = 0.2
_H = 16
_C = 16
_BLK = 1024


_P1 = 3.1416015625
_P2 = -8.907169103622437e-06
_P3 = -1.7412276065442711e-09
_P4 = 1.2434497875801753e-13
_INVPI = 0.3183098861837907


def _fast_sin(x):
    n = jnp.round(x * _INVPI)
    r = (((x - n * _P1) - n * _P2) - n * _P3) - n * _P4
    t = r * r
    q = ((2.7525562e-6 * t - 1.98408500e-4) * t + 8.3333310e-3) * t - 1.6666667e-1
    s = r + r * t * q
    m = n * 0.5
    fr = m - jnp.floor(m)
    return s * (1.0 - 4.0 * fr)


def _edge_mlp_body(xin_ref, v_ref, w0_ref, b0_ref, w1_ref, b1_ref,
                   w2_ref, b2_ref, sel_ref, msg_ref):
    xin = xin_ref[...]
    h = jnp.dot(xin, w0_ref[...], preferred_element_type=jnp.float32)
    h = _fast_sin(_OMEGA * (h + b0_ref[...]))
    h = jnp.dot(h, w1_ref[...], preferred_element_type=jnp.float32)
    h = _fast_sin(_OMEGA * (h + b1_ref[...]))
    h = jnp.dot(h, w2_ref[...], preferred_element_type=jnp.float32)
    h = _fast_sin(_OMEGA * (h + b2_ref[...]))
    vt = jnp.tile(v_ref[...], (1, _C))
    msg_ref[...] = jnp.dot(h * vt, sel_ref[...],
                           preferred_element_type=jnp.float32)


def _edge_mlp(xin, v, p, e_pad):
    w0 = p['s0W']
    w0bd = np.zeros((64, 16 * _C), np.float32)
    w1bd = np.zeros((16 * _C, 16 * _C), np.float32)
    w2bd = np.zeros((16 * _C, 16 * _C), np.float32)
    sel = np.zeros((16 * _C, _C), np.float32)
    w0bd = jnp.zeros((64, 16 * _C), jnp.float32)
    w1bd = jnp.zeros((16 * _C, 16 * _C), jnp.float32)
    w2bd = jnp.zeros((16 * _C, 16 * _C), jnp.float32)
    sel = jnp.zeros((16 * _C, _C), jnp.float32)
    for c in range(_C):
        w0bd = w0bd.at[c * 4:(c + 1) * 4, c * 16:(c + 1) * 16].set(w0.T)
        w1bd = w1bd.at[c * 16:(c + 1) * 16, c * 16:(c + 1) * 16].set(p['s1W'].T)
        w2bd = w2bd.at[c * 16:(c + 1) * 16, c * 16:(c + 1) * 16].set(p['s2W'].T)
        sel = sel.at[c * 16:(c + 1) * 16, c].set(1.0)
    b0t = jnp.tile(p['s0b'], _C)
    b1t = jnp.tile(p['s1b'], _C)
    b2t = jnp.tile(p['s2b'], _C)

    grid = (e_pad // _BLK,)
    full = lambda i: (0, 0)
    msg = pl.pallas_call(
        _edge_mlp_body,
        grid=grid,
        in_specs=[
            pl.BlockSpec((_BLK, 64), lambda i: (i, 0)),
            pl.BlockSpec((_BLK, _H), lambda i: (i, 0)),
            pl.BlockSpec((64, 16 * _C), full),
            pl.BlockSpec((1, 16 * _C), lambda i: (0, 0)),
            pl.BlockSpec((16 * _C, 16 * _C), full),
            pl.BlockSpec((1, 16 * _C), lambda i: (0, 0)),
            pl.BlockSpec((16 * _C, 16 * _C), full),
            pl.BlockSpec((1, 16 * _C), lambda i: (0, 0)),
            pl.BlockSpec((16 * _C, _C), full),
        ],
        out_specs=pl.BlockSpec((_BLK, _C), lambda i: (i, 0)),
        out_shape=jax.ShapeDtypeStruct((e_pad, _C), jnp.float32),
    )(xin, v, w0bd, b0t[None], w1bd, b1t[None], w2bd, b2t[None], sel)
    return msg


def _conv_msgs(xin, src, p, x_parts, e_pad):
    linW, linb = p['linW'], p['linb']
    xw = sum(jnp.dot(xp, linW[o:o + xp.shape[1]]) for xp, o in x_parts)
    xb = sum(jnp.dot(xp, linb[o:o + xp.shape[1]]) for xp, o in x_parts)
    v = xw[src]
    b = xb[src]
    msg = _edge_mlp(xin, v, p, e_pad)
    return msg + b[:, None]


def kernel(x, pos, params, edge_index):
    src0, dst0 = edge_index[0], edge_index[1]
    loops = jnp.arange(_N, dtype=edge_index.dtype)
    src = jnp.concatenate((src0, loops))
    dst = jnp.concatenate((dst0, loops))
    maskf = jnp.concatenate(((src0 != dst0).astype(jnp.float32),
                             jnp.ones((_N,), jnp.float32)))
    e = src.shape[0]
    e_pad = ((e + _BLK - 1) // _BLK) * _BLK
    pad = e_pad - e
    src_p = jnp.pad(src, (0, pad))
    rel = pos[dst] - pos[src]
    sq = jnp.sum(rel * rel, axis=1)
    rho = jnp.where(sq > 0, jnp.sqrt(jnp.where(sq > 0, sq, 1.0)), 0.0)
    x0, y0, z0c = rel[:, 0], rel[:, 1], rel[:, 2]
    xy_zero = (x0 == 0) & (y0 == 0)
    theta = jnp.arctan2(y0, jnp.where(xy_zero, 1.0, x0))
    phi = jnp.arcsin(z0c / jnp.where(rho == 0, 1.0, rho))
    ch = jnp.arange(_C, dtype=jnp.float32)
    feats4 = jnp.stack((rho, theta / jnp.pi, phi / jnp.pi), axis=1)
    xin = jnp.concatenate(
        (jnp.broadcast_to(feats4[:, None, :], (e, _C, 3)),
         jnp.broadcast_to(ch[None, :, None], (e, _C, 1))), axis=2
    ).reshape(e, 64)
    xin = jnp.pad(xin, ((0, pad), (0, 0)))

    p1, p2, p3 = params['conv1'], params['conv2'], params['conv3']

    def agg_of(msg, p):
        m = (msg[:e] * maskf[:, None])
        return jnp.zeros((_N, _C), jnp.float32).at[dst].add(m) + p['bias']

    x1 = jax.nn.selu(agg_of(_conv_msgs(xin, src_p, p1, [(x, 0)], e_pad), p1))
    x2 = jax.nn.selu(agg_of(
        _conv_msgs(xin, src_p, p2, [(x, 0), (x1, 16)], e_pad), p2))
    x3 = jax.nn.selu(agg_of(
        _conv_msgs(xin, src_p, p3, [(x, 0), (x1, 16), (x2, 32)], e_pad), p3))
    return x + _BETA * x3
